# Initial kernel scaffold; baseline (speedup 1.0000x reference)
import jax
import jax.numpy as jnp
from jax.experimental import pallas as pl


def kernel(timesteps, T, L, time_emb, pos_emb, type_emb):
    raise NotImplementedError("write your pallas kernel here")


# speedup vs baseline: 1.2669x; 1.2669x over previous
import functools

import jax
import jax.numpy as jnp
from jax import lax
from jax.experimental import pallas as pl
from jax.experimental.pallas import tpu as pltpu
from jax.experimental.pallas import tpu_sc as plsc

D = 128
NLANE = 16
NVEC = D // NLANE


def _make_sc_kernel(B, T, L):
    info = plsc.get_sparse_core_info()
    NC, NS = info.num_cores, info.num_subcores
    NW = NC * NS
    assert B % NW == 0
    b_per_w = B // NW

    CHUNK_T = 40
    assert T % CHUNK_T == 0
    n_chunks = T // CHUNK_T
    CHUNK_L = 3 * CHUNK_T

    mesh = plsc.VectorSubcoreMesh(core_axis_name="c", subcore_axis_name="s")

    @functools.partial(
        pl.kernel,
        out_type=jax.ShapeDtypeStruct((B, L, D), jnp.float32),
        mesh=mesh,
        scratch_types=[
            pltpu.VMEM((L, D), jnp.float32),
            pltpu.VMEM((3, D), jnp.float32),
            pltpu.VMEM((T,), jnp.int32),
            pltpu.VMEM((CHUNK_T, D), jnp.float32),
            pltpu.VMEM((CHUNK_L, D), jnp.float32),
            pltpu.SemaphoreType.DMA,
        ],
    )
    def sc_kernel(ts_hbm, time_hbm, pos_hbm, type_hbm, out_hbm,
                  comb_v, type_v, idx_v, gath_v, outst_v, sem):
        wid = lax.axis_index("s") * NC + lax.axis_index("c")

        pltpu.sync_copy(pos_hbm, comb_v)
        pltpu.sync_copy(type_hbm, type_v)
        tv = [[type_v[s, pl.ds(NLANE * j, NLANE)] for j in range(NVEC)]
              for s in range(3)]

        def comb_body(t, _):
            for s in range(3):
                row = 3 * t + s
                for j in range(NVEC):
                    sl = pl.ds(NLANE * j, NLANE)
                    comb_v[row, sl] = comb_v[row, sl] + tv[s][j]
            return 0

        lax.fori_loop(0, T, comb_body, 0)

        def batch_body(i, _):
            b = wid * b_per_w + i
            pltpu.sync_copy(ts_hbm.at[b], idx_v)

            def chunk_body(c, _):
                pltpu.async_copy(
                    time_hbm.at[idx_v.at[pl.ds(c * CHUNK_T, CHUNK_T)]],
                    gath_v, sem).wait()

                def t_body(t, _):
                    for j in range(NVEC):
                        sl = pl.ds(NLANE * j, NLANE)
                        g = gath_v[t, sl]
                        for s in range(3):
                            row = 3 * t + s
                            outst_v[row, sl] = g + comb_v[c * CHUNK_L + row, sl]
                    return 0

                lax.fori_loop(0, CHUNK_T, t_body, 0)
                pltpu.sync_copy(outst_v,
                                out_hbm.at[b, pl.ds(c * CHUNK_L, CHUNK_L)])
                return 0

            lax.fori_loop(0, n_chunks, chunk_body, 0)
            return 0

        lax.fori_loop(0, b_per_w, batch_body, 0)

    return sc_kernel


def kernel(timesteps, T, L, time_emb, pos_emb, type_emb):
    B, T_s = timesteps.shape
    L_s = pos_emb.shape[0]
    ts32 = timesteps.astype(jnp.int32)
    fn = _make_sc_kernel(B, T_s, L_s)
    return fn(ts32, time_emb, pos_emb, type_emb)


# speedup vs baseline: 3.4746x; 2.7426x over previous
import functools

import jax
import jax.numpy as jnp
from jax import lax
from jax.experimental import pallas as pl
from jax.experimental.pallas import tpu as pltpu
from jax.experimental.pallas import tpu_sc as plsc

D = 128
NLANE = 16
NVEC = D // NLANE


def _make_sc_kernel(B, T, L):
    info = plsc.get_sparse_core_info()
    NC, NS = info.num_cores, info.num_subcores
    NW = NC * NS
    assert B % NW == 0
    b_per_w = B // NW

    CHUNK_T = 40
    assert T % CHUNK_T == 0
    n_chunks = T // CHUNK_T
    CHUNK_L = 3 * CHUNK_T

    mesh = plsc.VectorSubcoreMesh(core_axis_name="c", subcore_axis_name="s")

    @functools.partial(
        pl.kernel,
        out_type=jax.ShapeDtypeStruct((B, L, D), jnp.float32),
        mesh=mesh,
        scratch_types=[
            pltpu.VMEM((L, D), jnp.float32),
            pltpu.VMEM((3, D), jnp.float32),
            pltpu.VMEM((T,), jnp.int32),
            pltpu.VMEM((2, CHUNK_T, D), jnp.float32),
            pltpu.VMEM((2, CHUNK_L, D), jnp.float32),
            pltpu.SemaphoreType.DMA,
            pltpu.SemaphoreType.DMA,
            pltpu.SemaphoreType.DMA,
            pltpu.SemaphoreType.DMA,
        ],
    )
    def sc_kernel(ts_hbm, time_hbm, pos_hbm, type_hbm, out_hbm,
                  comb_v, type_v, idx_v, gath_v, outst_v,
                  gsem0, gsem1, ssem0, ssem1):
        wid = lax.axis_index("s") * NC + lax.axis_index("c")

        pltpu.sync_copy(pos_hbm, comb_v)
        pltpu.sync_copy(type_hbm, type_v)
        tv = [[type_v[s, pl.ds(NLANE * j, NLANE)] for j in range(NVEC)]
              for s in range(3)]

        def comb_body(t, _):
            for s in range(3):
                row = 3 * t + s
                for j in range(NVEC):
                    sl = pl.ds(NLANE * j, NLANE)
                    comb_v[row, sl] = comb_v[row, sl] + tv[s][j]
            return 0

        lax.fori_loop(0, T, comb_body, 0)

        gsems = (gsem0, gsem1)
        ssems = (ssem0, ssem1)

        def batch_body(i, _):
            b = wid * b_per_w + i
            pltpu.sync_copy(ts_hbm.at[b], idx_v)

            def start_gather(c):
                p = c % 2
                return pltpu.async_copy(
                    time_hbm.at[idx_v.at[pl.ds(c * CHUNK_T, CHUNK_T)]],
                    gath_v.at[p], gsems[p])

            gathers = [start_gather(0), start_gather(1)]
            stores = [None, None]
            for c in range(n_chunks):
                p = c % 2
                gathers[p].wait()
                if stores[p] is not None:
                    stores[p].wait()

                def t_body(t, _, c=c, p=p):
                    for j in range(NVEC):
                        sl = pl.ds(NLANE * j, NLANE)
                        g = gath_v[p, t, sl]
                        for s in range(3):
                            row = 3 * t + s
                            outst_v[p, row, sl] = (
                                g + comb_v[c * CHUNK_L + row, sl])
                    return 0

                lax.fori_loop(0, CHUNK_T, t_body, 0)
                if c + 2 < n_chunks:
                    gathers[p] = start_gather(c + 2)
                stores[p] = pltpu.async_copy(
                    outst_v.at[p],
                    out_hbm.at[b, pl.ds(c * CHUNK_L, CHUNK_L)], ssems[p])
            stores[0].wait()
            stores[1].wait()
            return 0

        lax.fori_loop(0, b_per_w, batch_body, 0)

    return sc_kernel


def kernel(timesteps, T, L, time_emb, pos_emb, type_emb):
    B, T_s = timesteps.shape
    L_s = pos_emb.shape[0]
    ts32 = timesteps.astype(jnp.int32)
    fn = _make_sc_kernel(B, T_s, L_s)
    return fn(ts32, time_emb, pos_emb, type_emb)
